# baseline (speedup 1.0000x reference)
import functools

import jax
import jax.numpy as jnp
from jax import lax
from jax.experimental import pallas as pl
from jax.experimental.pallas import tpu as pltpu
from jax.experimental.pallas import tpu_sc as plsc

BATCH = 16384
LATENT = 32
NITEMS = 1000000
SROWS = (NITEMS + 127) // 128
RBLK = 896
GRID1 = (SROWS + RBLK - 1) // RBLK

NW = 32
BPW = BATCH // NW
NCHUNK = 4
CHUNK = BPW // NCHUNK


def _score_body(w_ref, b_ref, emb_ref, out_ref):
    x = emb_ref[...]
    s = jax.lax.dot_general(
        w_ref[...], x, (((1,), (0,)), ((), ())),
        preferred_element_type=jnp.float32)
    out_ref[...] = s.reshape(RBLK, 128) + b_ref[0, 0]


def _gather_body(idx_hbm, sc_hbm, out_hbm, idx_v, rows_v, dat_v, out_v, sem):
    wid = lax.axis_index("s") * 2 + lax.axis_index("c")
    base = wid * BPW

    pltpu.sync_copy(idx_hbm.at[pl.ds(base, BPW)], idx_v)

    def row_body(g, carry):
        iv = idx_v[pl.ds(g * 16, 16)]
        rows_v[pl.ds(g * 16, 16)] = iv >> 7
        return carry

    lax.fori_loop(0, BPW // 16, row_body, 0)

    copies = [
        pltpu.async_copy(
            sc_hbm.at[rows_v.at[pl.ds(c * CHUNK, CHUNK)]],
            dat_v.at[c],
            sem,
        )
        for c in range(NCHUNK)
    ]
    for cp in copies:
        cp.wait()

    lane = lax.iota(jnp.int32, 16)

    for c in range(NCHUNK):
        cvec = jnp.full((16,), c, jnp.int32)

        def ext_body(g, carry, c=c, cvec=cvec):
            iv = idx_v[pl.ds(c * CHUNK + g * 16, 16)]
            rvec = g * 16 + lane
            lvec = iv & 127
            vals = plsc.load_gather(dat_v, [cvec, rvec, lvec])
            out_v[pl.ds(c * CHUNK + g * 16, 16)] = vals
            return carry

        lax.fori_loop(0, CHUNK // 16, ext_body, 0)

    pltpu.sync_copy(out_v, out_hbm.at[pl.ds(base, BPW)])


@jax.jit
def kernel(item_indices, item_emb, ffn_w, ffn_b):
    wcol = ffn_w.astype(jnp.float32)
    bmat = ffn_b.reshape(1, 1).astype(jnp.float32)

    scores = pl.pallas_call(
        _score_body,
        grid=(GRID1,),
        in_specs=[
            pl.BlockSpec((1, LATENT), lambda i: (0, 0)),
            pl.BlockSpec((1, 1), lambda i: (0, 0)),
            pl.BlockSpec((LATENT, RBLK * 128), lambda i: (0, i)),
        ],
        out_specs=pl.BlockSpec((RBLK, 128), lambda i: (i, 0)),
        out_shape=jax.ShapeDtypeStruct((SROWS, 128), jnp.float32),
    )(wcol, bmat, item_emb.T)

    run = pl.kernel(
        _gather_body,
        mesh=plsc.VectorSubcoreMesh(core_axis_name="c", subcore_axis_name="s"),
        out_type=jax.ShapeDtypeStruct((BATCH,), jnp.float32),
        compiler_params=pltpu.CompilerParams(needs_layout_passes=False),
        scratch_types=[
            pltpu.VMEM((BPW,), jnp.int32),
            pltpu.VMEM((BPW,), jnp.int32),
            pltpu.VMEM((NCHUNK, CHUNK, 128), jnp.float32),
            pltpu.VMEM((BPW,), jnp.float32),
            pltpu.SemaphoreType.DMA,
        ],
    )
    out = run(item_indices.astype(jnp.int32), scores)
    return out.reshape(BATCH, 1)


# speedup vs baseline: 1.0164x; 1.0164x over previous
import functools

import jax
import jax.numpy as jnp
from jax import lax
from jax.experimental import pallas as pl
from jax.experimental.pallas import tpu as pltpu
from jax.experimental.pallas import tpu_sc as plsc

BATCH = 16384
LATENT = 32
NITEMS = 1000000
SROWS = (NITEMS + 127) // 128
RBLK = 672
GRID1 = (SROWS + RBLK - 1) // RBLK

NW = 32
BPW = BATCH // NW
NCHUNK = 4
CHUNK = BPW // NCHUNK


def _score_body(w_ref, b_ref, emb_ref, out_ref):
    x = emb_ref[...]
    s = jax.lax.dot_general(
        w_ref[...], x, (((1,), (0,)), ((), ())),
        preferred_element_type=jnp.float32)
    out_ref[...] = s.reshape(RBLK, 128) + b_ref[0, 0]


def _gather_body(idx_hbm, sc_hbm, out_hbm, idx_v, rows_v, dat_v, out_v, sem):
    wid = lax.axis_index("s") * 2 + lax.axis_index("c")
    base = wid * BPW

    pltpu.sync_copy(idx_hbm.at[pl.ds(base, BPW)], idx_v)

    def row_body(g, carry):
        iv = idx_v[pl.ds(g * 16, 16)]
        rows_v[pl.ds(g * 16, 16)] = iv >> 7
        return carry

    lax.fori_loop(0, BPW // 16, row_body, 0)

    copies = [
        pltpu.async_copy(
            sc_hbm.at[rows_v.at[pl.ds(c * CHUNK, CHUNK)]],
            dat_v.at[c],
            sem,
        )
        for c in range(NCHUNK)
    ]
    for cp in copies:
        cp.wait()

    lane = lax.iota(jnp.int32, 16)

    for c in range(NCHUNK):
        cvec = jnp.full((16,), c, jnp.int32)

        def ext_body(g, carry, c=c, cvec=cvec):
            iv = idx_v[pl.ds(c * CHUNK + g * 16, 16)]
            rvec = g * 16 + lane
            lvec = iv & 127
            vals = plsc.load_gather(dat_v, [cvec, rvec, lvec])
            out_v[pl.ds(c * CHUNK + g * 16, 16)] = vals
            return carry

        lax.fori_loop(0, CHUNK // 16, ext_body, 0)

    pltpu.sync_copy(out_v, out_hbm.at[pl.ds(base, BPW)])


@jax.jit
def kernel(item_indices, item_emb, ffn_w, ffn_b):
    wcol = ffn_w.astype(jnp.float32)
    bmat = ffn_b.reshape(1, 1).astype(jnp.float32)

    scores = pl.pallas_call(
        _score_body,
        grid=(GRID1,),
        in_specs=[
            pl.BlockSpec((1, LATENT), lambda i: (0, 0)),
            pl.BlockSpec((1, 1), lambda i: (0, 0)),
            pl.BlockSpec((LATENT, RBLK * 128), lambda i: (0, i)),
        ],
        out_specs=pl.BlockSpec((RBLK, 128), lambda i: (i, 0)),
        out_shape=jax.ShapeDtypeStruct((SROWS, 128), jnp.float32),
    )(wcol, bmat, item_emb.T)

    run = pl.kernel(
        _gather_body,
        mesh=plsc.VectorSubcoreMesh(core_axis_name="c", subcore_axis_name="s"),
        out_type=jax.ShapeDtypeStruct((BATCH,), jnp.float32),
        compiler_params=pltpu.CompilerParams(needs_layout_passes=False),
        scratch_types=[
            pltpu.VMEM((BPW,), jnp.int32),
            pltpu.VMEM((BPW,), jnp.int32),
            pltpu.VMEM((NCHUNK, CHUNK, 128), jnp.float32),
            pltpu.VMEM((BPW,), jnp.float32),
            pltpu.SemaphoreType.DMA,
        ],
    )
    out = run(item_indices.astype(jnp.int32), scores)
    return out.reshape(BATCH, 1)


# speedup vs baseline: 1.0214x; 1.0049x over previous
import functools

import jax
import jax.numpy as jnp
from jax import lax
from jax.experimental import pallas as pl
from jax.experimental.pallas import tpu as pltpu
from jax.experimental.pallas import tpu_sc as plsc

BATCH = 16384
LATENT = 32
NITEMS = 1000000
SROWS = (NITEMS + 127) // 128
RBLK = 512
GRID1 = (SROWS + RBLK - 1) // RBLK

NW = 32
BPW = BATCH // NW
NCHUNK = 4
CHUNK = BPW // NCHUNK


def _score_body(w_ref, b_ref, emb_ref, out_ref):
    x = emb_ref[...]
    s = jax.lax.dot_general(
        w_ref[...], x, (((1,), (0,)), ((), ())),
        preferred_element_type=jnp.float32)
    out_ref[...] = s.reshape(RBLK, 128) + b_ref[0, 0]


def _gather_body(idx_hbm, sc_hbm, out_hbm, idx_v, rows_v, dat_v, out_v, sem):
    wid = lax.axis_index("s") * 2 + lax.axis_index("c")
    base = wid * BPW

    pltpu.sync_copy(idx_hbm.at[pl.ds(base, BPW)], idx_v)

    def row_body(g, carry):
        iv = idx_v[pl.ds(g * 16, 16)]
        rows_v[pl.ds(g * 16, 16)] = iv >> 7
        return carry

    lax.fori_loop(0, BPW // 16, row_body, 0)

    copies = [
        pltpu.async_copy(
            sc_hbm.at[rows_v.at[pl.ds(c * CHUNK, CHUNK)]],
            dat_v.at[c],
            sem,
        )
        for c in range(NCHUNK)
    ]
    for cp in copies:
        cp.wait()

    lane = lax.iota(jnp.int32, 16)

    for c in range(NCHUNK):
        cvec = jnp.full((16,), c, jnp.int32)

        def ext_body(g, carry, c=c, cvec=cvec):
            iv = idx_v[pl.ds(c * CHUNK + g * 16, 16)]
            rvec = g * 16 + lane
            lvec = iv & 127
            vals = plsc.load_gather(dat_v, [cvec, rvec, lvec])
            out_v[pl.ds(c * CHUNK + g * 16, 16)] = vals
            return carry

        lax.fori_loop(0, CHUNK // 16, ext_body, 0)

    pltpu.sync_copy(out_v, out_hbm.at[pl.ds(base, BPW)])


@jax.jit
def kernel(item_indices, item_emb, ffn_w, ffn_b):
    wcol = ffn_w.astype(jnp.float32)
    bmat = ffn_b.reshape(1, 1).astype(jnp.float32)

    scores = pl.pallas_call(
        _score_body,
        grid=(GRID1,),
        in_specs=[
            pl.BlockSpec((1, LATENT), lambda i: (0, 0)),
            pl.BlockSpec((1, 1), lambda i: (0, 0)),
            pl.BlockSpec((LATENT, RBLK * 128), lambda i: (0, i)),
        ],
        out_specs=pl.BlockSpec((RBLK, 128), lambda i: (i, 0)),
        out_shape=jax.ShapeDtypeStruct((SROWS, 128), jnp.float32),
    )(wcol, bmat, item_emb.T)

    run = pl.kernel(
        _gather_body,
        mesh=plsc.VectorSubcoreMesh(core_axis_name="c", subcore_axis_name="s"),
        out_type=jax.ShapeDtypeStruct((BATCH,), jnp.float32),
        compiler_params=pltpu.CompilerParams(needs_layout_passes=False),
        scratch_types=[
            pltpu.VMEM((BPW,), jnp.int32),
            pltpu.VMEM((BPW,), jnp.int32),
            pltpu.VMEM((NCHUNK, CHUNK, 128), jnp.float32),
            pltpu.VMEM((BPW,), jnp.float32),
            pltpu.SemaphoreType.DMA,
        ],
    )
    out = run(item_indices.astype(jnp.int32), scores)
    return out.reshape(BATCH, 1)


# speedup vs baseline: 1.0301x; 1.0085x over previous
import jax
import jax.numpy as jnp
from jax import lax
from jax.experimental import pallas as pl
from jax.experimental.pallas import tpu as pltpu
from jax.experimental.pallas import tpu_sc as plsc

BATCH = 16384
LATENT = 32
NITEMS = 1000000
SROWS = (NITEMS + 127) // 128
RBLK = 448
GRID1 = (SROWS + RBLK - 1) // RBLK

NW = 32
BPW = BATCH // NW
NCHUNK = 4
CHUNK = BPW // NCHUNK


def _score_body(w_ref, b_ref, emb_ref, out_ref):
    x = emb_ref[...]
    s = jax.lax.dot_general(
        w_ref[...], x, (((1,), (0,)), ((), ())),
        preferred_element_type=jnp.float32)
    out_ref[...] = s.reshape(RBLK, 128) + b_ref[0, 0]


def _gather_body(idx_hbm, sc_hbm, out_hbm, idx_v, rows_v, dat_v, out_v, sem):
    wid = lax.axis_index("s") * 2 + lax.axis_index("c")
    base = wid * BPW

    pltpu.sync_copy(idx_hbm.at[pl.ds(base, BPW)], idx_v)

    def row_body(g, carry):
        iv = idx_v[pl.ds(g * 16, 16)]
        rows_v[pl.ds(g * 16, 16)] = iv >> 7
        return carry

    lax.fori_loop(0, BPW // 16, row_body, 0)

    copies = [
        pltpu.async_copy(
            sc_hbm.at[rows_v.at[pl.ds(c * CHUNK, CHUNK)]],
            dat_v.at[c],
            sem,
        )
        for c in range(NCHUNK)
    ]
    for cp in copies:
        cp.wait()

    lane = lax.iota(jnp.int32, 16)

    for c in range(NCHUNK):
        cvec = jnp.full((16,), c, jnp.int32)

        def ext_body(g, carry, c=c, cvec=cvec):
            iv = idx_v[pl.ds(c * CHUNK + g * 16, 16)]
            rvec = g * 16 + lane
            lvec = iv & 127
            vals = plsc.load_gather(dat_v, [cvec, rvec, lvec])
            out_v[pl.ds(c * CHUNK + g * 16, 16)] = vals
            return carry

        lax.fori_loop(0, CHUNK // 16, ext_body, 0)

    pltpu.sync_copy(out_v, out_hbm.at[pl.ds(base, BPW)])


@jax.jit
def kernel(item_indices, item_emb, ffn_w, ffn_b):
    wcol = ffn_w.astype(jnp.float32)
    bmat = ffn_b.reshape(1, 1).astype(jnp.float32)

    scores = pl.pallas_call(
        _score_body,
        grid=(GRID1,),
        in_specs=[
            pl.BlockSpec((1, LATENT), lambda i: (0, 0)),
            pl.BlockSpec((1, 1), lambda i: (0, 0)),
            pl.BlockSpec((LATENT, RBLK * 128), lambda i: (0, i)),
        ],
        out_specs=pl.BlockSpec((RBLK, 128), lambda i: (i, 0)),
        out_shape=jax.ShapeDtypeStruct((SROWS, 128), jnp.float32),
    )(wcol, bmat, item_emb.T)

    run = pl.kernel(
        _gather_body,
        mesh=plsc.VectorSubcoreMesh(core_axis_name="c", subcore_axis_name="s"),
        out_type=jax.ShapeDtypeStruct((BATCH,), jnp.float32),
        compiler_params=pltpu.CompilerParams(needs_layout_passes=False),
        scratch_types=[
            pltpu.VMEM((BPW,), jnp.int32),
            pltpu.VMEM((BPW,), jnp.int32),
            pltpu.VMEM((NCHUNK, CHUNK, 128), jnp.float32),
            pltpu.VMEM((BPW,), jnp.float32),
            pltpu.SemaphoreType.DMA,
        ],
    )
    out = run(item_indices.astype(jnp.int32), scores)
    return out.reshape(BATCH, 1)
